# baseline (speedup 1.0000x reference)
import functools

import jax
import jax.numpy as jnp
from jax import lax
from jax.experimental import pallas as pl
from jax.experimental.pallas import tpu as pltpu
from jax.experimental.pallas import tpu_sc as plsc

H = 50
B = 16384
D = 64
SCALE = 8.0
CHUNK = 128
LANES = 16
NBUF = 8
K = NBUF - 2
OBUF = 4
TJ = B // CHUNK
NSEG = D // LANES


@functools.cache
def _build():
    info = plsc.get_sparse_core_info()
    nc, ns = info.num_cores, info.num_subcores
    nw = nc * ns
    n_chunks = H * TJ
    per_w = n_chunks // nw
    assert per_w % NBUF == 0

    mesh = plsc.VectorSubcoreMesh(core_axis_name="c", subcore_axis_name="s")

    @functools.partial(
        pl.kernel,
        mesh=mesh,
        compiler_params=pltpu.CompilerParams(
            use_tc_tiling_on_sc=False, needs_layout_passes=False
        ),
        out_type=jax.ShapeDtypeStruct((H, D // 8, TJ, 8 * CHUNK), jnp.float32),
        scratch_types=(
            [pltpu.VMEM((per_w, CHUNK), jnp.int32)]
            + [pltpu.VMEM((CHUNK, D), jnp.float32)] * NBUF
            + [pltpu.VMEM((D * CHUNK,), jnp.float32)] * OBUF
            + [pltpu.SemaphoreType.DMA] * (NBUF + OBUF)
        ),
    )
    def k(ids_hbm, table_hbm, out_hbm, idx_all, *rest):
        rows = rest[:NBUF]
        obuf = rest[NBUF:NBUF + OBUF]
        gsem = rest[NBUF + OBUF:2 * NBUF + OBUF]
        ssem = rest[2 * NBUF + OBUF:]

        wid = lax.axis_index("s") * nc + lax.axis_index("c")
        cid0 = wid * per_w

        pltpu.sync_copy(ids_hbm.at[pl.ds(cid0, per_w)], idx_all)

        for c in range(K):
            pltpu.async_copy(table_hbm.at[idx_all.at[c]], rows[c], gsem[c])

        iota = lax.iota(jnp.int32, LANES)
        seg_base = [
            (((iota + c * LANES) >> 3) * (8 * CHUNK))
            + (((iota + c * LANES) & 7) * CHUNK)
            for c in range(NSEG)
        ]

        def outer(t, carry):
            for b in range(NBUF):
                g = t * NBUF + b
                sf = (b + K) % NBUF

                @pl.when(g + K < per_w)
                def _fire_gather():
                    pltpu.async_copy(
                        table_hbm.at[idx_all.at[g + K]], rows[sf], gsem[sf]
                    )

                pltpu.make_async_copy(
                    table_hbm.at[idx_all.at[g]], rows[b], gsem[b]
                ).wait()

                ob = b % OBUF
                cid = cid0 + g
                h = cid // TJ
                tj = cid - h * TJ

                def _wait_store():
                    for ti in range(D // 8):
                        pltpu.make_async_copy(
                            obuf[ob].at[pl.ds(ti * 8 * CHUNK, 8 * CHUNK)],
                            out_hbm.at[h, ti, tj],
                            ssem[ob],
                        ).wait()

                if b >= OBUF:
                    _wait_store()
                else:
                    pl.when(t >= 1)(_wait_store)

                def tpose_row(r, c2):
                    bi = jnp.full((LANES,), r, jnp.int32)
                    for c in range(NSEG):
                        v = rows[b][r, pl.ds(c * LANES, LANES)] * SCALE
                        plsc.store_scatter(obuf[ob], [seg_base[c] + bi], v)
                    return c2

                lax.fori_loop(0, CHUNK, tpose_row, 0)

                for ti in range(D // 8):
                    pltpu.async_copy(
                        obuf[ob].at[pl.ds(ti * 8 * CHUNK, 8 * CHUNK)],
                        out_hbm.at[h, ti, tj],
                        ssem[ob],
                    )
            return carry

        lax.fori_loop(0, per_w // NBUF, outer, 0)

        for ob in range(OBUF):
            for ti in range(D // 8):
                pltpu.make_async_copy(
                    obuf[ob].at[pl.ds(ti * 8 * CHUNK, 8 * CHUNK)],
                    out_hbm.at[0, ti, 0],
                    ssem[ob],
                ).wait()

    return k


def kernel(input_ids, table):
    ids2d = input_ids.T.reshape(H * TJ, CHUNK).astype(jnp.int32)
    out4d = _build()(ids2d, table)
    out5d = out4d.reshape(H, D // 8, TJ, 8, CHUNK)
    return out5d.transpose(2, 4, 0, 1, 3).reshape(B, H, D)


# speedup vs baseline: 1.0131x; 1.0131x over previous
import functools

import jax
import jax.numpy as jnp
from jax import lax
from jax.experimental import pallas as pl
from jax.experimental.pallas import tpu as pltpu
from jax.experimental.pallas import tpu_sc as plsc

H = 50
B = 16384
D = 64
SCALE = 8.0
CHUNK = 128
LANES = 16
NBUF = 8
K = NBUF - 2
OBUF = 4
TJ = B // CHUNK
NSEG = D // LANES


@functools.cache
def _build():
    info = plsc.get_sparse_core_info()
    nc, ns = info.num_cores, info.num_subcores
    nw = nc * ns
    n_chunks = H * TJ
    per_w = n_chunks // nw
    assert per_w % NBUF == 0

    mesh = plsc.VectorSubcoreMesh(core_axis_name="c", subcore_axis_name="s")

    @functools.partial(
        pl.kernel,
        mesh=mesh,
        compiler_params=pltpu.CompilerParams(
            use_tc_tiling_on_sc=False, needs_layout_passes=False
        ),
        out_type=jax.ShapeDtypeStruct((H, D // 8, TJ, 8 * CHUNK), jnp.float32),
        scratch_types=(
            [pltpu.VMEM((per_w, CHUNK), jnp.int32)]
            + [pltpu.VMEM((CHUNK, D), jnp.float32)] * NBUF
            + [pltpu.VMEM((D * CHUNK,), jnp.float32)] * OBUF
            + [pltpu.SemaphoreType.DMA] * (NBUF + OBUF)
        ),
    )
    def k(ids_hbm, table_hbm, out_hbm, idx_all, *rest):
        rows = rest[:NBUF]
        obuf = rest[NBUF:NBUF + OBUF]
        gsem = rest[NBUF + OBUF:2 * NBUF + OBUF]
        ssem = rest[2 * NBUF + OBUF:]

        wid = lax.axis_index("s") * nc + lax.axis_index("c")
        cid0 = wid * per_w

        pltpu.sync_copy(ids_hbm.at[pl.ds(cid0, per_w)], idx_all)

        for c in range(K):
            pltpu.async_copy(table_hbm.at[idx_all.at[c]], rows[c], gsem[c])

        iota = lax.iota(jnp.int32, LANES)
        seg_base = [
            (((iota + c * LANES) >> 3) * (8 * CHUNK))
            + (((iota + c * LANES) & 7) * CHUNK)
            for c in range(NSEG)
        ]

        def outer(t, carry):
            for b in range(NBUF):
                g = t * NBUF + b
                sf = (b + K) % NBUF

                @pl.when(g + K < per_w)
                def _fire_gather():
                    pltpu.async_copy(
                        table_hbm.at[idx_all.at[g + K]], rows[sf], gsem[sf]
                    )

                pltpu.make_async_copy(
                    table_hbm.at[idx_all.at[g]], rows[b], gsem[b]
                ).wait()

                ob = b % OBUF
                cid = cid0 + g
                h = cid // TJ
                tj = cid - h * TJ

                def _wait_store():
                    for ti in range(D // 8):
                        pltpu.make_async_copy(
                            obuf[ob].at[pl.ds(ti * 8 * CHUNK, 8 * CHUNK)],
                            out_hbm.at[h, ti, tj],
                            ssem[ob],
                        ).wait()

                if b >= OBUF:
                    _wait_store()
                else:
                    pl.when(t >= 1)(_wait_store)

                def tpose_grp(g, c2):
                    r0 = g * 16
                    for rr in range(16):
                        r = r0 + rr
                        for c in range(NSEG):
                            v = rows[b][r, pl.ds(c * LANES, LANES)] * SCALE
                            plsc.store_scatter(obuf[ob], [seg_base[c] + r], v)
                    return c2

                lax.fori_loop(0, CHUNK // 16, tpose_grp, 0)

                for ti in range(D // 8):
                    pltpu.async_copy(
                        obuf[ob].at[pl.ds(ti * 8 * CHUNK, 8 * CHUNK)],
                        out_hbm.at[h, ti, tj],
                        ssem[ob],
                    )
            return carry

        lax.fori_loop(0, per_w // NBUF, outer, 0)

        for ob in range(OBUF):
            for ti in range(D // 8):
                pltpu.make_async_copy(
                    obuf[ob].at[pl.ds(ti * 8 * CHUNK, 8 * CHUNK)],
                    out_hbm.at[0, ti, 0],
                    ssem[ob],
                ).wait()

    return k


def kernel(input_ids, table):
    ids2d = input_ids.T.reshape(H * TJ, CHUNK).astype(jnp.int32)
    out4d = _build()(ids2d, table)
    out5d = out4d.reshape(H, D // 8, TJ, 8, CHUNK)
    return out5d.transpose(2, 4, 0, 1, 3).reshape(B, H, D)


# speedup vs baseline: 1.5238x; 1.5041x over previous
import functools

import jax
import jax.numpy as jnp
from jax import lax
from jax.experimental import pallas as pl
from jax.experimental.pallas import tpu as pltpu
from jax.experimental.pallas import tpu_sc as plsc

H = 50
B = 16384
D = 64
SCALE = 8.0
CHUNK = 128
LANES = 16
NBUF = 8
K = NBUF - 2
OBUF = 4
OS = CHUNK + 8
TJ = B // CHUNK
NSEG = D // LANES


@functools.cache
def _build():
    info = plsc.get_sparse_core_info()
    nc, ns = info.num_cores, info.num_subcores
    nw = nc * ns
    n_chunks = H * TJ
    per_w = n_chunks // nw
    assert per_w % NBUF == 0

    mesh = plsc.VectorSubcoreMesh(core_axis_name="c", subcore_axis_name="s")

    @functools.partial(
        pl.kernel,
        mesh=mesh,
        compiler_params=pltpu.CompilerParams(
            use_tc_tiling_on_sc=False, needs_layout_passes=False
        ),
        out_type=jax.ShapeDtypeStruct((H, D // 8, TJ, 8 * CHUNK), jnp.float32),
        scratch_types=(
            [pltpu.VMEM((per_w, CHUNK), jnp.int32)]
            + [pltpu.VMEM((CHUNK, D), jnp.float32)] * NBUF
            + [pltpu.VMEM((D * OS,), jnp.float32)] * OBUF
            + [pltpu.SemaphoreType.DMA] * (NBUF + OBUF)
        ),
    )
    def k(ids_hbm, table_hbm, out_hbm, idx_all, *rest):
        rows = rest[:NBUF]
        obuf = rest[NBUF:NBUF + OBUF]
        gsem = rest[NBUF + OBUF:2 * NBUF + OBUF]
        ssem = rest[2 * NBUF + OBUF:]

        wid = lax.axis_index("s") * nc + lax.axis_index("c")
        cid0 = wid * per_w

        pltpu.sync_copy(ids_hbm.at[pl.ds(cid0, per_w)], idx_all)

        for c in range(K):
            pltpu.async_copy(table_hbm.at[idx_all.at[c]], rows[c], gsem[c])

        iota = lax.iota(jnp.int32, LANES)
        seg_base = [(iota + c * LANES) * OS for c in range(NSEG)]

        def outer(t, carry):
            for b in range(NBUF):
                g = t * NBUF + b
                sf = (b + K) % NBUF

                @pl.when(g + K < per_w)
                def _fire_gather():
                    pltpu.async_copy(
                        table_hbm.at[idx_all.at[g + K]], rows[sf], gsem[sf]
                    )

                pltpu.make_async_copy(
                    table_hbm.at[idx_all.at[g]], rows[b], gsem[b]
                ).wait()

                ob = b % OBUF
                cid = cid0 + g
                h = cid // TJ
                tj = cid - h * TJ

                def _wait_store():
                    def waits(ti, c2):
                        for di in range(8):
                            pltpu.make_async_copy(
                                obuf[ob].at[pl.ds((ti * 8 + di) * OS, CHUNK)],
                                out_hbm.at[h, ti, tj, pl.ds(di * CHUNK, CHUNK)],
                                ssem[ob],
                            ).wait()
                        return c2

                    lax.fori_loop(0, D // 8, waits, 0)

                if b >= OBUF:
                    _wait_store()
                else:
                    pl.when(t >= 1)(_wait_store)

                def tpose_grp(g, c2):
                    r0 = g * 16
                    for rr in range(16):
                        r = r0 + rr
                        for c in range(NSEG):
                            v = rows[b][r, pl.ds(c * LANES, LANES)] * SCALE
                            plsc.store_scatter(obuf[ob], [seg_base[c] + r], v)
                    return c2

                lax.fori_loop(0, CHUNK // 16, tpose_grp, 0)

                def issue_stores(ti, c2):
                    for di in range(8):
                        pltpu.async_copy(
                            obuf[ob].at[pl.ds((ti * 8 + di) * OS, CHUNK)],
                            out_hbm.at[h, ti, tj, pl.ds(di * CHUNK, CHUNK)],
                            ssem[ob],
                        )
                    return c2

                lax.fori_loop(0, D // 8, issue_stores, 0)
            return carry

        lax.fori_loop(0, per_w // NBUF, outer, 0)

        for ob in range(OBUF):
            def drain(ti, c2, ob=ob):
                for di in range(8):
                    pltpu.make_async_copy(
                        obuf[ob].at[pl.ds((ti * 8 + di) * OS, CHUNK)],
                        out_hbm.at[0, ti, 0, pl.ds(di * CHUNK, CHUNK)],
                        ssem[ob],
                    ).wait()
                return c2

            lax.fori_loop(0, D // 8, drain, 0)

    return k


def kernel(input_ids, table):
    ids2d = input_ids.T.reshape(H * TJ, CHUNK).astype(jnp.int32)
    out4d = _build()(ids2d, table)
    out5d = out4d.reshape(H, D // 8, TJ, 8, CHUNK)
    return out5d.transpose(2, 4, 0, 1, 3).reshape(B, H, D)


# speedup vs baseline: 1.5686x; 1.0294x over previous
import functools

import jax
import jax.numpy as jnp
from jax import lax
from jax.experimental import pallas as pl
from jax.experimental.pallas import tpu as pltpu
from jax.experimental.pallas import tpu_sc as plsc

H = 50
B = 16384
D = 64
SCALE = 8.0
CHUNK = 128
LANES = 16
NBUF = 8
K = NBUF - 2
OBUF = 4
OS = CHUNK + 8
TJ = B // CHUNK
NSEG = D // LANES


@functools.cache
def _build():
    info = plsc.get_sparse_core_info()
    nc, ns = info.num_cores, info.num_subcores
    nw = nc * ns
    n_chunks = H * TJ
    per_w = n_chunks // nw
    assert per_w % NBUF == 0

    mesh = plsc.VectorSubcoreMesh(core_axis_name="c", subcore_axis_name="s")

    @functools.partial(
        pl.kernel,
        mesh=mesh,
        compiler_params=pltpu.CompilerParams(
            use_tc_tiling_on_sc=False, needs_layout_passes=False
        ),
        out_type=jax.ShapeDtypeStruct((H, D // 8, TJ, 8, CHUNK), jnp.float32),
        scratch_types=(
            [pltpu.VMEM((per_w, CHUNK), jnp.int32)]
            + [pltpu.VMEM((CHUNK, D), jnp.float32)] * NBUF
            + [pltpu.VMEM((D // 8, 8, OS), jnp.float32)] * OBUF
            + [pltpu.SemaphoreType.DMA] * (NBUF + OBUF)
        ),
    )
    def k(ids_hbm, table_hbm, out_hbm, idx_all, *rest):
        rows = rest[:NBUF]
        obuf = rest[NBUF:NBUF + OBUF]
        gsem = rest[NBUF + OBUF:2 * NBUF + OBUF]
        ssem = rest[2 * NBUF + OBUF:]

        wid = lax.axis_index("s") * nc + lax.axis_index("c")
        cid0 = wid * per_w

        pltpu.sync_copy(ids_hbm.at[pl.ds(cid0, per_w)], idx_all)

        for c in range(K):
            pltpu.async_copy(table_hbm.at[idx_all.at[c]], rows[c], gsem[c])

        iota = lax.iota(jnp.int32, LANES)
        seg_ti = [(iota + c * LANES) >> 3 for c in range(NSEG)]
        seg_di = [(iota + c * LANES) & 7 for c in range(NSEG)]

        def outer(t, carry):
            for b in range(NBUF):
                g = t * NBUF + b
                sf = (b + K) % NBUF

                @pl.when(g + K < per_w)
                def _fire_gather():
                    pltpu.async_copy(
                        table_hbm.at[idx_all.at[g + K]], rows[sf], gsem[sf]
                    )

                pltpu.make_async_copy(
                    table_hbm.at[idx_all.at[g]], rows[b], gsem[b]
                ).wait()

                ob = b % OBUF
                cid = cid0 + g
                h = cid // TJ
                tj = cid - h * TJ

                def _wait_store():
                    pltpu.make_async_copy(
                        obuf[ob].at[:, :, pl.ds(0, CHUNK)],
                        out_hbm.at[h, :, tj],
                        ssem[ob],
                    ).wait()

                if b >= OBUF:
                    _wait_store()
                else:
                    pl.when(t >= 1)(_wait_store)

                def tpose_grp(g, c2):
                    r0 = g * 16
                    for rr in range(16):
                        r = r0 + rr
                        for c in range(NSEG):
                            v = rows[b][r, pl.ds(c * LANES, LANES)] * SCALE
                            plsc.store_scatter(
                                obuf[ob],
                                [seg_ti[c], seg_di[c], jnp.full((LANES,), r, jnp.int32)],
                                v,
                            )
                    return c2

                lax.fori_loop(0, CHUNK // 16, tpose_grp, 0)

                pltpu.async_copy(
                    obuf[ob].at[:, :, pl.ds(0, CHUNK)],
                    out_hbm.at[h, :, tj],
                    ssem[ob],
                )
            return carry

        lax.fori_loop(0, per_w // NBUF, outer, 0)

        for ob in range(OBUF):
            pltpu.make_async_copy(
                obuf[ob].at[:, :, pl.ds(0, CHUNK)],
                out_hbm.at[0, :, 0],
                ssem[ob],
            ).wait()

    return k


def kernel(input_ids, table):
    ids2d = input_ids.T.reshape(H * TJ, CHUNK).astype(jnp.int32)
    out4d = _build()(ids2d, table)
    return out4d.transpose(2, 4, 0, 1, 3).reshape(B, H, D)
